# Initial kernel scaffold; baseline (speedup 1.0000x reference)
import jax
import jax.numpy as jnp
from jax.experimental import pallas as pl


def kernel(input, table, W, b):
    raise NotImplementedError("write your pallas kernel here")


# speedup vs baseline: 26.1545x; 26.1545x over previous
import functools

import jax
import jax.numpy as jnp
from jax import lax
from jax.experimental import pallas as pl
from jax.experimental.pallas import tpu as pltpu
from jax.experimental.pallas import tpu_sc as plsc

MIDDLE = 30
FEATURES = 26
EMBED_DIM = 32
BATCH = 16384

NC = 2
NS = 16
NW = NC * NS
BF = BATCH * FEATURES
RPW = BF // NW
IDXW = 128
CHUNK = 512
KSUB = CHUNK // IDXW
NSTEP = RPW // CHUNK

DENSE_BLK = 1024


@functools.partial(
    pl.kernel,
    mesh=plsc.VectorSubcoreMesh(core_axis_name="c", subcore_axis_name="s"),
    out_type=jax.ShapeDtypeStruct((BF, EMBED_DIM), jnp.float32),
    scratch_types=[
        pltpu.VMEM((NSTEP * KSUB, IDXW), jnp.int32),
        pltpu.VMEM((CHUNK, EMBED_DIM), jnp.float32),
        pltpu.SemaphoreType.DMA,
    ],
    compiler_params=pltpu.CompilerParams(use_tc_tiling_on_sc=False),
)
def _sc_gather(idx_hbm, table_hbm, out_hbm, idx_v, rows_v, sem):
    wid = lax.axis_index("s") * NC + lax.axis_index("c")
    base = wid * RPW
    pltpu.sync_copy(idx_hbm.at[wid], idx_v)

    def step(c, carry):
        copies = []
        for j in range(KSUB):
            copies.append(
                pltpu.async_copy(
                    table_hbm.at[idx_v.at[c * KSUB + j]],
                    rows_v.at[pl.ds(j * IDXW, IDXW)],
                    sem,
                )
            )
        for cpy in copies:
            cpy.wait()
        off = pl.multiple_of(base + c * CHUNK, CHUNK)
        pltpu.sync_copy(rows_v, out_hbm.at[pl.ds(off, CHUNK)])
        return carry

    lax.fori_loop(0, NSTEP, step, 0)


def _dense_body(x_ref, w_ref, b_ref, o_ref):
    acc = jnp.dot(x_ref[...], w_ref[...], preferred_element_type=jnp.float32)
    o_ref[...] = jnp.maximum(acc + b_ref[...], 0.0)


def kernel(input, table, W, b):
    idx = input.astype(jnp.int32).reshape(NW, NSTEP * KSUB, IDXW)
    gathered = _sc_gather(idx, table)
    x = gathered.reshape(BATCH, FEATURES * EMBED_DIM)
    out = pl.pallas_call(
        _dense_body,
        grid=(BATCH // DENSE_BLK,),
        in_specs=[
            pl.BlockSpec((DENSE_BLK, FEATURES * EMBED_DIM), lambda i: (i, 0)),
            pl.BlockSpec((FEATURES * EMBED_DIM, MIDDLE), lambda i: (0, 0)),
            pl.BlockSpec((1, MIDDLE), lambda i: (0, 0)),
        ],
        out_specs=pl.BlockSpec((DENSE_BLK, MIDDLE), lambda i: (i, 0)),
        out_shape=jax.ShapeDtypeStruct((BATCH, MIDDLE), jnp.float32),
    )(x, W, b.reshape(1, MIDDLE))
    return out
